# Initial kernel scaffold; baseline (speedup 1.0000x reference)
import jax
import jax.numpy as jnp
from jax.experimental import pallas as pl


def kernel(x, edge_index, W1, b1, g1, be1, W2, b2, g2, be2, Wh1, bh1, Wh2, bh2):
    raise NotImplementedError("write your pallas kernel here")


# speedup vs baseline: 5.7162x; 5.7162x over previous
import functools

import jax
import jax.numpy as jnp
from jax import lax
from jax.experimental import pallas as pl
from jax.experimental.pallas import tpu as pltpu
from jax.experimental.pallas import tpu_sc as plsc

N = 50000
NPAD = 51200
E = 800000
EROWS, ECOLS = 6400, 125
ROWS_PER_W = EROWS // 32
BLK_ROWS = 8
NBLK = ROWS_PER_W // BLK_ROWS
ROWS_PER_T = EROWS // 16
NBLK_S = ROWS_PER_T // BLK_ROWS
NPT = NPAD // 16
R = 1600
NFC = 8
FCW = 16

def _mesh():
    return plsc.VectorSubcoreMesh(core_axis_name="c", subcore_axis_name="s")


def _sc_count_body(dst_hbm, zeros1_hbm, cnt_hbm, dstv, onesv, acc):
    cid = lax.axis_index("c")
    sid = lax.axis_index("s")
    wid = cid * 16 + sid
    for i in range(8):
        onesv[pl.ds(i * 16, 16)] = jnp.ones((16,), jnp.float32)
    pltpu.sync_copy(zeros1_hbm, acc.at[pl.ds(sid * NPT, NPT)])
    plsc.subcore_barrier()

    def body(b, _):
        r0 = wid * ROWS_PER_W + b * BLK_ROWS
        pltpu.sync_copy(dst_hbm.at[pl.ds(r0, BLK_ROWS), :], dstv)
        for j in range(BLK_ROWS):
            pltpu.sync_copy(onesv.at[pl.ds(0, ECOLS)],
                            acc.at[dstv.at[j]], add=True)
        return 0

    lax.fori_loop(0, NBLK, body, 0)
    plsc.subcore_barrier()
    for c in range(2):
        @pl.when(cid == c)
        def _():
            pltpu.sync_copy(acc.at[pl.ds(sid * NPT, NPT)],
                            cnt_hbm.at[c].at[pl.ds(sid * NPT, NPT)])


def _sc_count(dst2d, zeros1):
    return pl.kernel(
        _sc_count_body,
        out_type=jax.ShapeDtypeStruct((2, NPAD), jnp.float32),
        mesh=_mesh(),
        scratch_types=[
            pltpu.VMEM((BLK_ROWS, ECOLS), jnp.int32),
            pltpu.VMEM((128,), jnp.float32),
            pltpu.VMEM_SHARED((NPAD,), jnp.float32),
        ],
    )(dst2d, zeros1)


def _sc_scatter_body(*refs):
    tables = refs[0:8]
    src_hbm, dst_hbm, zeros16_hbm = refs[8], refs[9], refs[10]
    outs = refs[11:19]
    srcv, dstv, rowsv, acc = refs[19], refs[20], refs[21], refs[22]
    cid = lax.axis_index("c")
    sid = lax.axis_index("s")

    for c in range(2):
        @pl.when(cid == c)
        def _():
            for k in range(4):
                fc = 4 * c + k
                table, out = tables[fc], outs[fc]
                pltpu.sync_copy(zeros16_hbm, acc.at[pl.ds(sid * NPT, NPT), :])
                plsc.subcore_barrier()

                def body(b, _):
                    r0 = sid * ROWS_PER_T + b * BLK_ROWS
                    pltpu.sync_copy(src_hbm.at[pl.ds(r0, BLK_ROWS), :], srcv)
                    pltpu.sync_copy(dst_hbm.at[pl.ds(r0, BLK_ROWS), :], dstv)
                    for j in range(BLK_ROWS):
                        pltpu.sync_copy(
                            table.at[srcv.at[j]],
                            rowsv.at[pl.ds(j * ECOLS, ECOLS), :])
                    for j in range(BLK_ROWS):
                        pltpu.sync_copy(
                            rowsv.at[pl.ds(j * ECOLS, ECOLS), :],
                            acc.at[dstv.at[j]], add=True)
                    return 0

                lax.fori_loop(0, NBLK_S, body, 0)
                plsc.subcore_barrier()
                pltpu.sync_copy(acc.at[pl.ds(sid * NPT, NPT), :],
                                out.at[pl.ds(sid * NPT, NPT), :])
                plsc.subcore_barrier()


def _sc_scatter(hs_fc, src2d, dst2d, zeros16):
    out_t = tuple(jax.ShapeDtypeStruct((NPAD, FCW), jnp.float32)
                  for _ in range(NFC))
    return pl.kernel(
        _sc_scatter_body,
        out_type=out_t,
        mesh=_mesh(),
        compiler_params=pltpu.CompilerParams(use_tc_tiling_on_sc=False),
        scratch_types=[
            pltpu.VMEM((BLK_ROWS, ECOLS), jnp.int32),
            pltpu.VMEM((BLK_ROWS, ECOLS), jnp.int32),
            pltpu.VMEM((BLK_ROWS * ECOLS, FCW), jnp.float32),
            pltpu.VMEM_SHARED((NPAD, FCW), jnp.float32),
        ],
    )(*hs_fc, src2d, dst2d, zeros16)


def _tc_a_body(x_ref, cnt_ref, w1_ref, dinv_ref, *hs_refs):
    cnt = cnt_ref[...]
    deg = cnt[:, 0:1] + cnt[:, 1:2] + 1.0
    dinv = lax.rsqrt(deg)
    dinv_ref[...] = dinv
    h = jnp.dot(x_ref[...], w1_ref[...], preferred_element_type=jnp.float32)
    hs = h * dinv
    for k in range(NFC):
        hs_refs[k][...] = hs[:, k * FCW:(k + 1) * FCW]


def _tc_a(x_pad, cnt_t, W1):
    nblk = NPAD // R
    grid = (nblk,)
    in_specs = [
        pl.BlockSpec((R, 64), lambda i: (i, 0)),
        pl.BlockSpec((R, 2), lambda i: (i, 0)),
        pl.BlockSpec((64, 128), lambda i: (0, 0)),
    ]
    out_specs = [pl.BlockSpec((R, 1), lambda i: (i, 0))] + [
        pl.BlockSpec((R, FCW), lambda i: (i, 0)) for _ in range(NFC)]
    out_shape = [jax.ShapeDtypeStruct((NPAD, 1), jnp.float32)] + [
        jax.ShapeDtypeStruct((NPAD, FCW), jnp.float32) for _ in range(NFC)]
    res = pl.pallas_call(
        _tc_a_body, grid=grid, in_specs=in_specs,
        out_specs=out_specs, out_shape=out_shape,
    )(x_pad, cnt_t, W1)
    return res[0], tuple(res[1:])


def _tc_stats_body(*refs):
    s_refs, hs_refs = refs[0:8], refs[8:16]
    dinv_ref, params_ref = refs[16], refs[17]
    t_ref, sums_ref = refs[18], refs[19]
    i = pl.program_id(0)
    dinv = dinv_ref[...]
    b_row = params_ref[...][0:1, :]
    parts = [dinv * (s_refs[k][...] + hs_refs[k][...]) for k in range(NFC)]
    t = jnp.concatenate(parts, axis=1) + b_row
    t_ref[...] = t
    node = i * R + lax.broadcasted_iota(jnp.int32, (R, 1), 0)
    mask = (node < N).astype(jnp.float32)
    tm = t * mask
    p0 = jnp.sum(tm, axis=0, keepdims=True)
    p1 = jnp.sum(tm * t, axis=0, keepdims=True)

    @pl.when(i == 0)
    def _():
        sums_ref[...] = jnp.zeros((8, 128), jnp.float32)

    sums_ref[0:1, :] += p0
    sums_ref[1:2, :] += p1


def _tc_stats(s_fc, hs_fc, dinv, b):
    nblk = NPAD // R
    params = jnp.zeros((8, 128), jnp.float32).at[0].set(b)
    in_specs = (
        [pl.BlockSpec((R, FCW), lambda i: (i, 0)) for _ in range(NFC)] * 2
        + [pl.BlockSpec((R, 1), lambda i: (i, 0)),
           pl.BlockSpec((8, 128), lambda i: (0, 0))])
    out_specs = [pl.BlockSpec((R, 128), lambda i: (i, 0)),
                 pl.BlockSpec((8, 128), lambda i: (0, 0))]
    out_shape = [jax.ShapeDtypeStruct((NPAD, 128), jnp.float32),
                 jax.ShapeDtypeStruct((8, 128), jnp.float32)]
    return pl.pallas_call(
        _tc_stats_body, grid=(nblk,), in_specs=in_specs,
        out_specs=out_specs, out_shape=out_shape,
    )(*s_fc, *hs_fc, dinv, params)


def _bn_coefs(sums, params):
    mean = sums[0:1, :] / float(N)
    var = sums[1:2, :] / float(N) - mean * mean
    scale = params[0:1, :] * lax.rsqrt(var + 1e-5)
    shift = params[1:2, :] - mean * scale
    return scale, shift


def _tc_c_body(t_ref, sums_ref, dinv_ref, params_ref, w2_ref, *hs_refs):
    scale, shift = _bn_coefs(sums_ref[...], params_ref[...])
    r = jnp.maximum(t_ref[...] * scale + shift, 0.0)
    h2 = jnp.dot(r, w2_ref[...], preferred_element_type=jnp.float32)
    hs2 = h2 * dinv_ref[...]
    for k in range(NFC):
        hs_refs[k][...] = hs2[:, k * FCW:(k + 1) * FCW]


def _tc_c(t, sums, dinv, g, be, W2):
    nblk = NPAD // R
    params = jnp.zeros((8, 128), jnp.float32).at[0].set(g).at[1].set(be)
    in_specs = [
        pl.BlockSpec((R, 128), lambda i: (i, 0)),
        pl.BlockSpec((8, 128), lambda i: (0, 0)),
        pl.BlockSpec((R, 1), lambda i: (i, 0)),
        pl.BlockSpec((8, 128), lambda i: (0, 0)),
        pl.BlockSpec((128, 128), lambda i: (0, 0)),
    ]
    out_specs = [pl.BlockSpec((R, FCW), lambda i: (i, 0)) for _ in range(NFC)]
    out_shape = [jax.ShapeDtypeStruct((NPAD, FCW), jnp.float32)
                 for _ in range(NFC)]
    return tuple(pl.pallas_call(
        _tc_c_body, grid=(nblk,), in_specs=in_specs,
        out_specs=out_specs, out_shape=out_shape,
    )(t, sums, dinv, params, W2))


def _tc_e_body(t_ref, sums_ref, params_ref, wh1_ref, y_ref):
    params = params_ref[...]
    scale, shift = _bn_coefs(sums_ref[...], params)
    r = jnp.maximum(t_ref[...] * scale + shift, 0.0)
    hh = jnp.dot(r, wh1_ref[...], preferred_element_type=jnp.float32)
    hh = jnp.maximum(hh + params[2:3, 0:64], 0.0)
    z = jnp.sum(hh * params[4:5, 0:64], axis=1, keepdims=True) \
        + params[3:4, 0:1]
    y_ref[...] = jax.nn.sigmoid(z)


def _tc_e(t2, sums2, g2, be2, Wh1, bh1, Wh2, bh2):
    nblk = NPAD // R
    params = (jnp.zeros((8, 128), jnp.float32)
              .at[0].set(g2).at[1].set(be2)
              .at[2, 0:64].set(bh1)
              .at[3, 0].set(bh2[0])
              .at[4, 0:64].set(Wh2[:, 0]))
    in_specs = [
        pl.BlockSpec((R, 128), lambda i: (i, 0)),
        pl.BlockSpec((8, 128), lambda i: (0, 0)),
        pl.BlockSpec((8, 128), lambda i: (0, 0)),
        pl.BlockSpec((128, 64), lambda i: (0, 0)),
    ]
    return pl.pallas_call(
        _tc_e_body, grid=(nblk,), in_specs=in_specs,
        out_specs=pl.BlockSpec((R, 1), lambda i: (i, 0)),
        out_shape=jax.ShapeDtypeStruct((NPAD, 1), jnp.float32),
    )(t2, sums2, params, Wh1)


def kernel(x, edge_index, W1, b1, g1, be1, W2, b2, g2, be2, Wh1, bh1, Wh2, bh2):
    ei = edge_index.astype(jnp.int32)
    src2d = ei[0].reshape(EROWS, ECOLS)
    dst2d = ei[1].reshape(EROWS, ECOLS)
    x_pad = jnp.pad(x, ((0, NPAD - N), (0, 0)))
    zeros1 = jnp.zeros((NPT,), jnp.float32)
    zeros16 = jnp.zeros((NPT, FCW), jnp.float32)

    cnt = _sc_count(dst2d, zeros1)
    cnt_t = cnt.T
    dinv, hs1 = _tc_a(x_pad, cnt_t, W1)
    s1 = _sc_scatter(hs1, src2d, dst2d, zeros16)
    t1, sums1 = _tc_stats(s1, hs1, dinv, b1)
    hs2 = _tc_c(t1, sums1, dinv, g1, be1, W2)
    s2 = _sc_scatter(hs2, src2d, dst2d, zeros16)
    t2, sums2 = _tc_stats(s2, hs2, dinv, b2)
    y = _tc_e(t2, sums2, g2, be2, Wh1, bh1, Wh2, bh2)
    return y[:N]


# speedup vs baseline: 12.8427x; 2.2467x over previous
import functools

import jax
import jax.numpy as jnp
from jax import lax
from jax.experimental import pallas as pl
from jax.experimental.pallas import tpu as pltpu
from jax.experimental.pallas import tpu_sc as plsc

N = 50000
NPAD = 51200
E = 800000
EROWS, ECOLS = 6400, 125
ROWS_PER_W = EROWS // 32
BLK_ROWS = 8
NBLK = ROWS_PER_W // BLK_ROWS
ROWS_PER_T = EROWS // 16
NBLK_S = ROWS_PER_T // BLK_ROWS
NPT = NPAD // 16
R = 1600
NFC = 8
FCW = 16

def _mesh():
    return plsc.VectorSubcoreMesh(core_axis_name="c", subcore_axis_name="s")


def _sc_count_body(dst_hbm, zeros1_hbm, cnt_hbm, dstv, onesv, acc):
    cid = lax.axis_index("c")
    sid = lax.axis_index("s")
    wid = cid * 16 + sid
    for i in range(8):
        onesv[pl.ds(i * 16, 16)] = jnp.ones((16,), jnp.float32)
    pltpu.sync_copy(zeros1_hbm, acc.at[pl.ds(sid * NPT, NPT)])
    plsc.subcore_barrier()

    def body(b, _):
        r0 = wid * ROWS_PER_W + b * BLK_ROWS
        pltpu.sync_copy(dst_hbm.at[pl.ds(r0, BLK_ROWS), :], dstv)
        for j in range(BLK_ROWS):
            pltpu.sync_copy(onesv.at[pl.ds(0, ECOLS)],
                            acc.at[dstv.at[j]], add=True)
        return 0

    lax.fori_loop(0, NBLK, body, 0)
    plsc.subcore_barrier()
    for c in range(2):
        @pl.when(cid == c)
        def _():
            pltpu.sync_copy(acc.at[pl.ds(sid * NPT, NPT)],
                            cnt_hbm.at[c].at[pl.ds(sid * NPT, NPT)])


def _sc_count(dst2d, zeros1):
    return pl.kernel(
        _sc_count_body,
        out_type=jax.ShapeDtypeStruct((2, NPAD), jnp.float32),
        mesh=_mesh(),
        scratch_types=[
            pltpu.VMEM((BLK_ROWS, ECOLS), jnp.int32),
            pltpu.VMEM((128,), jnp.float32),
            pltpu.VMEM_SHARED((NPAD,), jnp.float32),
        ],
    )(dst2d, zeros1)


def _sc_scatter_body(*refs):
    tables = refs[0:8]
    src_hbm, dst_hbm, zeros16_hbm = refs[8], refs[9], refs[10]
    outs = refs[11:19]
    srcv, dstv, rowsv, acc = refs[19], refs[20], refs[21], refs[22]
    isem, dsem, gsem, ssem = refs[23], refs[24], refs[25], refs[26]
    cid = lax.axis_index("c")
    sid = lax.axis_index("s")

    def src_start(b, p):
        r0 = sid * ROWS_PER_T + b * BLK_ROWS
        pltpu.async_copy(src_hbm.at[pl.ds(r0, BLK_ROWS), :],
                         srcv.at[p], isem.at[p])

    def src_wait(p):
        pltpu.make_async_copy(src_hbm.at[pl.ds(0, BLK_ROWS), :],
                              srcv.at[p], isem.at[p]).wait()

    def dst_start(b, p):
        r0 = sid * ROWS_PER_T + b * BLK_ROWS
        pltpu.async_copy(dst_hbm.at[pl.ds(r0, BLK_ROWS), :],
                         dstv.at[p], dsem.at[p])

    def dst_wait(p):
        pltpu.make_async_copy(dst_hbm.at[pl.ds(0, BLK_ROWS), :],
                              dstv.at[p], dsem.at[p]).wait()

    def scat_drain(p):
        for j in range(BLK_ROWS):
            pltpu.make_async_copy(
                rowsv.at[p].at[pl.ds(j * ECOLS, ECOLS), :],
                acc.at[dstv.at[p].at[j]], ssem.at[p]).wait()

    def step(table, b, p):
        @pl.when(b >= 2)
        def _():
            scat_drain(p)
        dst_start(b, p)
        src_wait(p)
        gs = [pltpu.async_copy(
            table.at[srcv.at[p].at[j]],
            rowsv.at[p].at[pl.ds(j * ECOLS, ECOLS), :], gsem)
            for j in range(BLK_ROWS)]
        for g in gs:
            g.wait()

        @pl.when(b + 2 < NBLK_S)
        def _():
            src_start(b + 2, p)
        dst_wait(p)
        for j in range(BLK_ROWS):
            pltpu.async_copy(
                rowsv.at[p].at[pl.ds(j * ECOLS, ECOLS), :],
                acc.at[dstv.at[p].at[j]], ssem.at[p], add=True)

    for c in range(2):
        @pl.when(cid == c)
        def _():
            for k in range(4):
                fc = 4 * c + k
                table, out = tables[fc], outs[fc]
                pltpu.sync_copy(zeros16_hbm, acc.at[pl.ds(sid * NPT, NPT), :])
                plsc.subcore_barrier()
                src_start(0, 0)
                src_start(1, 1)

                def body(i, _):
                    step(table, i * 2, 0)
                    step(table, i * 2 + 1, 1)
                    return 0

                lax.fori_loop(0, NBLK_S // 2, body, 0)
                scat_drain(0)
                scat_drain(1)
                plsc.subcore_barrier()
                pltpu.sync_copy(acc.at[pl.ds(sid * NPT, NPT), :],
                                out.at[pl.ds(sid * NPT, NPT), :])
                plsc.subcore_barrier()


def _sc_scatter(hs_fc, src2d, dst2d, zeros16):
    out_t = tuple(jax.ShapeDtypeStruct((NPAD, FCW), jnp.float32)
                  for _ in range(NFC))
    return pl.kernel(
        _sc_scatter_body,
        out_type=out_t,
        mesh=_mesh(),
        compiler_params=pltpu.CompilerParams(use_tc_tiling_on_sc=False),
        scratch_types=[
            pltpu.VMEM((2, BLK_ROWS, ECOLS), jnp.int32),
            pltpu.VMEM((2, BLK_ROWS, ECOLS), jnp.int32),
            pltpu.VMEM((2, BLK_ROWS * ECOLS, FCW), jnp.float32),
            pltpu.VMEM_SHARED((NPAD, FCW), jnp.float32),
            pltpu.SemaphoreType.DMA((2,)),
            pltpu.SemaphoreType.DMA((2,)),
            pltpu.SemaphoreType.DMA,
            pltpu.SemaphoreType.DMA((2,)),
        ],
    )(*hs_fc, src2d, dst2d, zeros16)


def _tc_a_body(x_ref, cnt_ref, w1_ref, dinv_ref, *hs_refs):
    cnt = cnt_ref[...]
    deg = cnt[:, 0:1] + cnt[:, 1:2] + 1.0
    dinv = lax.rsqrt(deg)
    dinv_ref[...] = dinv
    h = jnp.dot(x_ref[...], w1_ref[...], preferred_element_type=jnp.float32)
    hs = h * dinv
    for k in range(NFC):
        hs_refs[k][...] = hs[:, k * FCW:(k + 1) * FCW]


def _tc_a(x_pad, cnt_t, W1):
    nblk = NPAD // R
    grid = (nblk,)
    in_specs = [
        pl.BlockSpec((R, 64), lambda i: (i, 0)),
        pl.BlockSpec((R, 2), lambda i: (i, 0)),
        pl.BlockSpec((64, 128), lambda i: (0, 0)),
    ]
    out_specs = [pl.BlockSpec((R, 1), lambda i: (i, 0))] + [
        pl.BlockSpec((R, FCW), lambda i: (i, 0)) for _ in range(NFC)]
    out_shape = [jax.ShapeDtypeStruct((NPAD, 1), jnp.float32)] + [
        jax.ShapeDtypeStruct((NPAD, FCW), jnp.float32) for _ in range(NFC)]
    res = pl.pallas_call(
        _tc_a_body, grid=grid, in_specs=in_specs,
        out_specs=out_specs, out_shape=out_shape,
    )(x_pad, cnt_t, W1)
    return res[0], tuple(res[1:])


def _tc_stats_body(*refs):
    s_refs, hs_refs = refs[0:8], refs[8:16]
    dinv_ref, params_ref = refs[16], refs[17]
    t_ref, sums_ref = refs[18], refs[19]
    i = pl.program_id(0)
    dinv = dinv_ref[...]
    b_row = params_ref[...][0:1, :]
    parts = [dinv * (s_refs[k][...] + hs_refs[k][...]) for k in range(NFC)]
    t = jnp.concatenate(parts, axis=1) + b_row
    t_ref[...] = t
    node = i * R + lax.broadcasted_iota(jnp.int32, (R, 1), 0)
    mask = (node < N).astype(jnp.float32)
    tm = t * mask
    p0 = jnp.sum(tm, axis=0, keepdims=True)
    p1 = jnp.sum(tm * t, axis=0, keepdims=True)

    @pl.when(i == 0)
    def _():
        sums_ref[...] = jnp.zeros((8, 128), jnp.float32)

    sums_ref[0:1, :] += p0
    sums_ref[1:2, :] += p1


def _tc_stats(s_fc, hs_fc, dinv, b):
    nblk = NPAD // R
    params = jnp.zeros((8, 128), jnp.float32).at[0].set(b)
    in_specs = (
        [pl.BlockSpec((R, FCW), lambda i: (i, 0)) for _ in range(NFC)] * 2
        + [pl.BlockSpec((R, 1), lambda i: (i, 0)),
           pl.BlockSpec((8, 128), lambda i: (0, 0))])
    out_specs = [pl.BlockSpec((R, 128), lambda i: (i, 0)),
                 pl.BlockSpec((8, 128), lambda i: (0, 0))]
    out_shape = [jax.ShapeDtypeStruct((NPAD, 128), jnp.float32),
                 jax.ShapeDtypeStruct((8, 128), jnp.float32)]
    return pl.pallas_call(
        _tc_stats_body, grid=(nblk,), in_specs=in_specs,
        out_specs=out_specs, out_shape=out_shape,
    )(*s_fc, *hs_fc, dinv, params)


def _bn_coefs(sums, params):
    mean = sums[0:1, :] / float(N)
    var = sums[1:2, :] / float(N) - mean * mean
    scale = params[0:1, :] * lax.rsqrt(var + 1e-5)
    shift = params[1:2, :] - mean * scale
    return scale, shift


def _tc_c_body(t_ref, sums_ref, dinv_ref, params_ref, w2_ref, *hs_refs):
    scale, shift = _bn_coefs(sums_ref[...], params_ref[...])
    r = jnp.maximum(t_ref[...] * scale + shift, 0.0)
    h2 = jnp.dot(r, w2_ref[...], preferred_element_type=jnp.float32)
    hs2 = h2 * dinv_ref[...]
    for k in range(NFC):
        hs_refs[k][...] = hs2[:, k * FCW:(k + 1) * FCW]


def _tc_c(t, sums, dinv, g, be, W2):
    nblk = NPAD // R
    params = jnp.zeros((8, 128), jnp.float32).at[0].set(g).at[1].set(be)
    in_specs = [
        pl.BlockSpec((R, 128), lambda i: (i, 0)),
        pl.BlockSpec((8, 128), lambda i: (0, 0)),
        pl.BlockSpec((R, 1), lambda i: (i, 0)),
        pl.BlockSpec((8, 128), lambda i: (0, 0)),
        pl.BlockSpec((128, 128), lambda i: (0, 0)),
    ]
    out_specs = [pl.BlockSpec((R, FCW), lambda i: (i, 0)) for _ in range(NFC)]
    out_shape = [jax.ShapeDtypeStruct((NPAD, FCW), jnp.float32)
                 for _ in range(NFC)]
    return tuple(pl.pallas_call(
        _tc_c_body, grid=(nblk,), in_specs=in_specs,
        out_specs=out_specs, out_shape=out_shape,
    )(t, sums, dinv, params, W2))


def _tc_e_body(t_ref, sums_ref, params_ref, wh1_ref, y_ref):
    params = params_ref[...]
    scale, shift = _bn_coefs(sums_ref[...], params)
    r = jnp.maximum(t_ref[...] * scale + shift, 0.0)
    hh = jnp.dot(r, wh1_ref[...], preferred_element_type=jnp.float32)
    hh = jnp.maximum(hh + params[2:3, 0:64], 0.0)
    z = jnp.sum(hh * params[4:5, 0:64], axis=1, keepdims=True) \
        + params[3:4, 0:1]
    y_ref[...] = jax.nn.sigmoid(z)


def _tc_e(t2, sums2, g2, be2, Wh1, bh1, Wh2, bh2):
    nblk = NPAD // R
    params = (jnp.zeros((8, 128), jnp.float32)
              .at[0].set(g2).at[1].set(be2)
              .at[2, 0:64].set(bh1)
              .at[3, 0].set(bh2[0])
              .at[4, 0:64].set(Wh2[:, 0]))
    in_specs = [
        pl.BlockSpec((R, 128), lambda i: (i, 0)),
        pl.BlockSpec((8, 128), lambda i: (0, 0)),
        pl.BlockSpec((8, 128), lambda i: (0, 0)),
        pl.BlockSpec((128, 64), lambda i: (0, 0)),
    ]
    return pl.pallas_call(
        _tc_e_body, grid=(nblk,), in_specs=in_specs,
        out_specs=pl.BlockSpec((R, 1), lambda i: (i, 0)),
        out_shape=jax.ShapeDtypeStruct((NPAD, 1), jnp.float32),
    )(t2, sums2, params, Wh1)


def kernel(x, edge_index, W1, b1, g1, be1, W2, b2, g2, be2, Wh1, bh1, Wh2, bh2):
    ei = edge_index.astype(jnp.int32)
    src2d = ei[0].reshape(EROWS, ECOLS)
    dst2d = ei[1].reshape(EROWS, ECOLS)
    x_pad = jnp.pad(x, ((0, NPAD - N), (0, 0)))
    zeros1 = jnp.zeros((NPT,), jnp.float32)
    zeros16 = jnp.zeros((NPT, FCW), jnp.float32)

    cnt = _sc_count(dst2d, zeros1)
    cnt_t = cnt.T
    dinv, hs1 = _tc_a(x_pad, cnt_t, W1)
    s1 = _sc_scatter(hs1, src2d, dst2d, zeros16)
    t1, sums1 = _tc_stats(s1, hs1, dinv, b1)
    hs2 = _tc_c(t1, sums1, dinv, g1, be1, W2)
    s2 = _sc_scatter(hs2, src2d, dst2d, zeros16)
    t2, sums2 = _tc_stats(s2, hs2, dinv, b2)
    y = _tc_e(t2, sums2, g2, be2, Wh1, bh1, Wh2, bh2)
    return y[:N]


# speedup vs baseline: 15.3861x; 1.1980x over previous
import functools

import jax
import jax.numpy as jnp
from jax import lax
from jax.experimental import pallas as pl
from jax.experimental.pallas import tpu as pltpu
from jax.experimental.pallas import tpu_sc as plsc

N = 50000
NPAD = 51200
E = 800000
EROWS, ECOLS = 6400, 125
ROWS_PER_W = EROWS // 32
BLK_ROWS = 8
NBLK = ROWS_PER_W // BLK_ROWS
ROWS_PER_T = EROWS // 16
NBLK_S = ROWS_PER_T // BLK_ROWS
NPT = NPAD // 16
R = 6400
NFC = 8
FCW = 16

def _mesh():
    return plsc.VectorSubcoreMesh(core_axis_name="c", subcore_axis_name="s")


def _sc_count_body(dst_hbm, zeros1_hbm, cnt_hbm, dstv, onesv, acc):
    cid = lax.axis_index("c")
    sid = lax.axis_index("s")
    wid = cid * 16 + sid
    for i in range(8):
        onesv[pl.ds(i * 16, 16)] = jnp.ones((16,), jnp.float32)
    pltpu.sync_copy(zeros1_hbm, acc.at[pl.ds(sid * NPT, NPT)])
    plsc.subcore_barrier()

    def body(b, _):
        r0 = wid * ROWS_PER_W + b * BLK_ROWS
        pltpu.sync_copy(dst_hbm.at[pl.ds(r0, BLK_ROWS), :], dstv)
        for j in range(BLK_ROWS):
            pltpu.sync_copy(onesv.at[pl.ds(0, ECOLS)],
                            acc.at[dstv.at[j]], add=True)
        return 0

    lax.fori_loop(0, NBLK, body, 0)
    plsc.subcore_barrier()
    for c in range(2):
        @pl.when(cid == c)
        def _():
            pltpu.sync_copy(acc.at[pl.ds(sid * NPT, NPT)],
                            cnt_hbm.at[c].at[pl.ds(sid * NPT, NPT)])


def _sc_count(dst2d, zeros1):
    return pl.kernel(
        _sc_count_body,
        out_type=jax.ShapeDtypeStruct((2, NPAD), jnp.float32),
        mesh=_mesh(),
        scratch_types=[
            pltpu.VMEM((BLK_ROWS, ECOLS), jnp.int32),
            pltpu.VMEM((128,), jnp.float32),
            pltpu.VMEM_SHARED((NPAD,), jnp.float32),
        ],
    )(dst2d, zeros1)


def _sc_scatter_body(table, src8_hbm, dst_hbm, zeros16_hbm, out,
                     srcv, dstv, rowsv, acc, isem, dsem, gsem, ssem):
    cid = lax.axis_index("c")
    sid = lax.axis_index("s")

    def src_start(fc, b, p):
        r0 = sid * ROWS_PER_T + b * BLK_ROWS
        pltpu.async_copy(src8_hbm.at[fc].at[pl.ds(r0, BLK_ROWS), :],
                         srcv.at[p], isem.at[p])

    def src_wait(p):
        pltpu.make_async_copy(src8_hbm.at[0].at[pl.ds(0, BLK_ROWS), :],
                              srcv.at[p], isem.at[p]).wait()

    def dst_start(b, p):
        r0 = sid * ROWS_PER_T + b * BLK_ROWS
        pltpu.async_copy(dst_hbm.at[pl.ds(r0, BLK_ROWS), :],
                         dstv.at[p], dsem.at[p])

    def dst_wait(p):
        pltpu.make_async_copy(dst_hbm.at[pl.ds(0, BLK_ROWS), :],
                              dstv.at[p], dsem.at[p]).wait()

    def scat_drain(p):
        for j in range(BLK_ROWS):
            pltpu.make_async_copy(
                rowsv.at[p].at[pl.ds(j * ECOLS, ECOLS), :],
                acc.at[dstv.at[p].at[j]], ssem.at[p]).wait()

    def step(fc, b, p):
        @pl.when(b >= 2)
        def _():
            scat_drain(p)
        dst_start(b, p)
        src_wait(p)
        gs = [pltpu.async_copy(
            table.at[srcv.at[p].at[j]],
            rowsv.at[p].at[pl.ds(j * ECOLS, ECOLS), :], gsem)
            for j in range(BLK_ROWS)]
        for g in gs:
            g.wait()

        @pl.when(b + 2 < NBLK_S)
        def _():
            src_start(fc, b + 2, p)
        dst_wait(p)
        for j in range(BLK_ROWS):
            pltpu.async_copy(
                rowsv.at[p].at[pl.ds(j * ECOLS, ECOLS), :],
                acc.at[dstv.at[p].at[j]], ssem.at[p], add=True)

    for c in range(2):
        @pl.when(cid == c)
        def _():
            for k in range(4):
                fc = 4 * c + k
                pltpu.sync_copy(zeros16_hbm, acc.at[pl.ds(sid * NPT, NPT), :])
                plsc.subcore_barrier()
                src_start(fc, 0, 0)
                src_start(fc, 1, 1)

                def body(i, _):
                    step(fc, i * 2, 0)
                    step(fc, i * 2 + 1, 1)
                    return 0

                lax.fori_loop(0, NBLK_S // 2, body, 0)
                scat_drain(0)
                scat_drain(1)
                plsc.subcore_barrier()
                pltpu.sync_copy(acc.at[pl.ds(sid * NPT, NPT), :],
                                out.at[pl.ds(sid * NPT, NPT), fc, :])
                plsc.subcore_barrier()


def _sc_scatter(hs_flat, src8, dst2d, zeros16):
    return pl.kernel(
        _sc_scatter_body,
        out_type=jax.ShapeDtypeStruct((NPAD, NFC, FCW), jnp.float32),
        mesh=_mesh(),
        compiler_params=pltpu.CompilerParams(use_tc_tiling_on_sc=False),
        scratch_types=[
            pltpu.VMEM((2, BLK_ROWS, ECOLS), jnp.int32),
            pltpu.VMEM((2, BLK_ROWS, ECOLS), jnp.int32),
            pltpu.VMEM((2, BLK_ROWS * ECOLS, FCW), jnp.float32),
            pltpu.VMEM_SHARED((NPAD, FCW), jnp.float32),
            pltpu.SemaphoreType.DMA((2,)),
            pltpu.SemaphoreType.DMA((2,)),
            pltpu.SemaphoreType.DMA,
            pltpu.SemaphoreType.DMA((2,)),
        ],
    )(hs_flat, src8, dst2d, zeros16)


def _tc_idx_body(src_ref, out_ref):
    fc = pl.program_id(0)
    out_ref[0] = src_ref[...] * 8 + fc


def _tc_idx(src2d):
    erb = 800
    return pl.pallas_call(
        _tc_idx_body, grid=(NFC, EROWS // erb),
        in_specs=[pl.BlockSpec((erb, ECOLS), lambda fc, j: (j, 0))],
        out_specs=pl.BlockSpec((1, erb, ECOLS), lambda fc, j: (fc, j, 0)),
        out_shape=jax.ShapeDtypeStruct((NFC, EROWS, ECOLS), jnp.int32),
    )(src2d)


def _tc_a_body(x_ref, cnt_ref, w1_ref, dinv_ref, hs_ref):
    cnt = cnt_ref[...]
    deg = cnt[:, 0:1] + cnt[:, 1:2] + 1.0
    dinv = lax.rsqrt(deg)
    dinv_ref[...] = dinv
    h = jnp.dot(x_ref[...], w1_ref[...], preferred_element_type=jnp.float32)
    hs_ref[...] = h * dinv


def _tc_a(x_pad, cnt_t, W1):
    nblk = NPAD // R
    grid = (nblk,)
    in_specs = [
        pl.BlockSpec((R, 64), lambda i: (i, 0)),
        pl.BlockSpec((R, 2), lambda i: (i, 0)),
        pl.BlockSpec((64, 128), lambda i: (0, 0)),
    ]
    out_specs = [pl.BlockSpec((R, 1), lambda i: (i, 0)),
                 pl.BlockSpec((R, 128), lambda i: (i, 0))]
    out_shape = [jax.ShapeDtypeStruct((NPAD, 1), jnp.float32),
                 jax.ShapeDtypeStruct((NPAD, 128), jnp.float32)]
    res = pl.pallas_call(
        _tc_a_body, grid=grid, in_specs=in_specs,
        out_specs=out_specs, out_shape=out_shape,
    )(x_pad, cnt_t, W1)
    return res[0], res[1]


def _tc_stats_body(s_ref, hs_ref, dinv_ref, params_ref, t_ref, sums_ref):
    i = pl.program_id(0)
    dinv = dinv_ref[...]
    b_row = params_ref[...][0:1, :]
    t = dinv * (s_ref[...] + hs_ref[...]) + b_row
    t_ref[...] = t
    node = i * R + lax.broadcasted_iota(jnp.int32, (R, 1), 0)
    mask = (node < N).astype(jnp.float32)
    tm = t * mask
    p0 = jnp.sum(tm, axis=0, keepdims=True)
    p1 = jnp.sum(tm * t, axis=0, keepdims=True)

    @pl.when(i == 0)
    def _():
        sums_ref[...] = jnp.zeros((8, 128), jnp.float32)

    sums_ref[0:1, :] += p0
    sums_ref[1:2, :] += p1


def _tc_stats(s, hs, dinv, b):
    nblk = NPAD // R
    params = jnp.zeros((8, 128), jnp.float32).at[0].set(b)
    in_specs = [pl.BlockSpec((R, 128), lambda i: (i, 0)),
                pl.BlockSpec((R, 128), lambda i: (i, 0)),
                pl.BlockSpec((R, 1), lambda i: (i, 0)),
                pl.BlockSpec((8, 128), lambda i: (0, 0))]
    out_specs = [pl.BlockSpec((R, 128), lambda i: (i, 0)),
                 pl.BlockSpec((8, 128), lambda i: (0, 0))]
    out_shape = [jax.ShapeDtypeStruct((NPAD, 128), jnp.float32),
                 jax.ShapeDtypeStruct((8, 128), jnp.float32)]
    return pl.pallas_call(
        _tc_stats_body, grid=(nblk,), in_specs=in_specs,
        out_specs=out_specs, out_shape=out_shape,
    )(s, hs, dinv, params)


def _bn_coefs(sums, params):
    mean = sums[0:1, :] / float(N)
    var = sums[1:2, :] / float(N) - mean * mean
    scale = params[0:1, :] * lax.rsqrt(var + 1e-5)
    shift = params[1:2, :] - mean * scale
    return scale, shift


def _tc_c_body(t_ref, sums_ref, dinv_ref, params_ref, w2_ref, hs_ref):
    scale, shift = _bn_coefs(sums_ref[...], params_ref[...])
    r = jnp.maximum(t_ref[...] * scale + shift, 0.0)
    h2 = jnp.dot(r, w2_ref[...], preferred_element_type=jnp.float32)
    hs_ref[...] = h2 * dinv_ref[...]


def _tc_c(t, sums, dinv, g, be, W2):
    nblk = NPAD // R
    params = jnp.zeros((8, 128), jnp.float32).at[0].set(g).at[1].set(be)
    in_specs = [
        pl.BlockSpec((R, 128), lambda i: (i, 0)),
        pl.BlockSpec((8, 128), lambda i: (0, 0)),
        pl.BlockSpec((R, 1), lambda i: (i, 0)),
        pl.BlockSpec((8, 128), lambda i: (0, 0)),
        pl.BlockSpec((128, 128), lambda i: (0, 0)),
    ]
    return pl.pallas_call(
        _tc_c_body, grid=(nblk,), in_specs=in_specs,
        out_specs=pl.BlockSpec((R, 128), lambda i: (i, 0)),
        out_shape=jax.ShapeDtypeStruct((NPAD, 128), jnp.float32),
    )(t, sums, dinv, params, W2)


def _tc_e_body(t_ref, sums_ref, params_ref, wh1_ref, y_ref):
    params = params_ref[...]
    scale, shift = _bn_coefs(sums_ref[...], params)
    r = jnp.maximum(t_ref[...] * scale + shift, 0.0)
    hh = jnp.dot(r, wh1_ref[...], preferred_element_type=jnp.float32)
    hh = jnp.maximum(hh + params[2:3, 0:64], 0.0)
    z = jnp.sum(hh * params[4:5, 0:64], axis=1, keepdims=True) \
        + params[3:4, 0:1]
    y_ref[...] = jax.nn.sigmoid(z)


def _tc_e(t2, sums2, g2, be2, Wh1, bh1, Wh2, bh2):
    nblk = NPAD // R
    params = (jnp.zeros((8, 128), jnp.float32)
              .at[0].set(g2).at[1].set(be2)
              .at[2, 0:64].set(bh1)
              .at[3, 0].set(bh2[0])
              .at[4, 0:64].set(Wh2[:, 0]))
    in_specs = [
        pl.BlockSpec((R, 128), lambda i: (i, 0)),
        pl.BlockSpec((8, 128), lambda i: (0, 0)),
        pl.BlockSpec((8, 128), lambda i: (0, 0)),
        pl.BlockSpec((128, 64), lambda i: (0, 0)),
    ]
    return pl.pallas_call(
        _tc_e_body, grid=(nblk,), in_specs=in_specs,
        out_specs=pl.BlockSpec((R, 1), lambda i: (i, 0)),
        out_shape=jax.ShapeDtypeStruct((NPAD, 1), jnp.float32),
    )(t2, sums2, params, Wh1)


def kernel(x, edge_index, W1, b1, g1, be1, W2, b2, g2, be2, Wh1, bh1, Wh2, bh2):
    ei = edge_index.astype(jnp.int32)
    src2d = ei[0].reshape(EROWS, ECOLS)
    dst2d = ei[1].reshape(EROWS, ECOLS)
    x_pad = jnp.pad(x, ((0, NPAD - N), (0, 0)))
    zeros1 = jnp.zeros((NPT,), jnp.float32)
    zeros16 = jnp.zeros((NPT, FCW), jnp.float32)

    src8 = _tc_idx(src2d)
    cnt = _sc_count(dst2d, zeros1)
    cnt_t = cnt.T
    dinv, hs1 = _tc_a(x_pad, cnt_t, W1)
    s1 = _sc_scatter(hs1.reshape(NPAD * NFC, FCW), src8, dst2d,
                     zeros16).reshape(NPAD, 128)
    t1, sums1 = _tc_stats(s1, hs1, dinv, b1)
    hs2 = _tc_c(t1, sums1, dinv, g1, be1, W2)
    s2 = _sc_scatter(hs2.reshape(NPAD * NFC, FCW), src8, dst2d,
                     zeros16).reshape(NPAD, 128)
    t2, sums2 = _tc_stats(s2, hs2, dinv, b2)
    y = _tc_e(t2, sums2, g2, be2, Wh1, bh1, Wh2, bh2)
    return y[:N]


# speedup vs baseline: 15.8707x; 1.0315x over previous
import functools

import jax
import jax.numpy as jnp
from jax import lax
from jax.experimental import pallas as pl
from jax.experimental.pallas import tpu as pltpu
from jax.experimental.pallas import tpu_sc as plsc

N = 50000
NPAD = 51200
E = 800000
EROWS, ECOLS = 6400, 125
ROWS_PER_W = EROWS // 32
BLK_ROWS = 10
CBLK = 8
NBLK = ROWS_PER_W // CBLK
ROWS_PER_T = EROWS // 16
NBLK_S = ROWS_PER_T // BLK_ROWS
NPT = NPAD // 16
R = 6400
NFC = 8
FCW = 16

def _mesh():
    return plsc.VectorSubcoreMesh(core_axis_name="c", subcore_axis_name="s")


def _sc_count_body(dst_hbm, zeros1_hbm, cnt_hbm, dstv, onesv, acc):
    cid = lax.axis_index("c")
    sid = lax.axis_index("s")
    wid = cid * 16 + sid
    for i in range(8):
        onesv[pl.ds(i * 16, 16)] = jnp.ones((16,), jnp.float32)
    pltpu.sync_copy(zeros1_hbm, acc.at[pl.ds(sid * NPT, NPT)])
    plsc.subcore_barrier()

    def body(b, _):
        r0 = wid * ROWS_PER_W + b * CBLK
        pltpu.sync_copy(dst_hbm.at[pl.ds(r0, CBLK), :], dstv)
        for j in range(CBLK):
            pltpu.sync_copy(onesv.at[pl.ds(0, ECOLS)],
                            acc.at[dstv.at[j]], add=True)
        return 0

    lax.fori_loop(0, NBLK, body, 0)
    plsc.subcore_barrier()
    for c in range(2):
        @pl.when(cid == c)
        def _():
            pltpu.sync_copy(acc.at[pl.ds(sid * NPT, NPT)],
                            cnt_hbm.at[c].at[pl.ds(sid * NPT, NPT)])


def _sc_count(dst2d, zeros1):
    return pl.kernel(
        _sc_count_body,
        out_type=jax.ShapeDtypeStruct((2, NPAD), jnp.float32),
        mesh=_mesh(),
        scratch_types=[
            pltpu.VMEM((CBLK, ECOLS), jnp.int32),
            pltpu.VMEM((128,), jnp.float32),
            pltpu.VMEM_SHARED((NPAD,), jnp.float32),
        ],
    )(dst2d, zeros1)


def _sc_scatter_body(table, src8_hbm, dst_hbm, zeros16_hbm, out,
                     srcv, dstv, rowsv, acc, isem, dsem, gsem, ssem):
    cid = lax.axis_index("c")
    sid = lax.axis_index("s")

    def src_start(fc, b, p):
        r0 = sid * ROWS_PER_T + b * BLK_ROWS
        pltpu.async_copy(src8_hbm.at[fc].at[pl.ds(r0, BLK_ROWS), :],
                         srcv.at[p], isem.at[p])

    def src_wait(p):
        pltpu.make_async_copy(src8_hbm.at[0].at[pl.ds(0, BLK_ROWS), :],
                              srcv.at[p], isem.at[p]).wait()

    def dst_start(b, p):
        r0 = sid * ROWS_PER_T + b * BLK_ROWS
        pltpu.async_copy(dst_hbm.at[pl.ds(r0, BLK_ROWS), :],
                         dstv.at[p], dsem.at[p])

    def dst_wait(p):
        pltpu.make_async_copy(dst_hbm.at[pl.ds(0, BLK_ROWS), :],
                              dstv.at[p], dsem.at[p]).wait()

    def scat_drain(p):
        for j in range(BLK_ROWS):
            pltpu.make_async_copy(
                rowsv.at[p].at[pl.ds(j * ECOLS, ECOLS), :],
                acc.at[dstv.at[p].at[j]], ssem.at[p]).wait()

    def step(fc, b, p):
        @pl.when(b >= 2)
        def _():
            scat_drain(p)
        dst_start(b, p)
        src_wait(p)
        gs = [pltpu.async_copy(
            table.at[srcv.at[p].at[j]],
            rowsv.at[p].at[pl.ds(j * ECOLS, ECOLS), :], gsem)
            for j in range(BLK_ROWS)]
        for g in gs:
            g.wait()

        @pl.when(b + 2 < NBLK_S)
        def _():
            src_start(fc, b + 2, p)
        dst_wait(p)
        for j in range(BLK_ROWS):
            pltpu.async_copy(
                rowsv.at[p].at[pl.ds(j * ECOLS, ECOLS), :],
                acc.at[dstv.at[p].at[j]], ssem.at[p], add=True)

    for c in range(2):
        @pl.when(cid == c)
        def _():
            for k in range(4):
                fc = 4 * c + k
                pltpu.sync_copy(zeros16_hbm, acc.at[pl.ds(sid * NPT, NPT), :])
                plsc.subcore_barrier()
                src_start(fc, 0, 0)
                src_start(fc, 1, 1)

                def body(i, _):
                    step(fc, i * 2, 0)
                    step(fc, i * 2 + 1, 1)
                    return 0

                lax.fori_loop(0, NBLK_S // 2, body, 0)
                scat_drain(0)
                scat_drain(1)
                plsc.subcore_barrier()
                pltpu.sync_copy(acc.at[pl.ds(sid * NPT, NPT), :],
                                out.at[pl.ds(sid * NPT, NPT), fc, :])
                plsc.subcore_barrier()


def _sc_scatter(hs_flat, src8, dst2d, zeros16):
    return pl.kernel(
        _sc_scatter_body,
        out_type=jax.ShapeDtypeStruct((NPAD, NFC, FCW), jnp.float32),
        mesh=_mesh(),
        compiler_params=pltpu.CompilerParams(use_tc_tiling_on_sc=False),
        scratch_types=[
            pltpu.VMEM((2, BLK_ROWS, ECOLS), jnp.int32),
            pltpu.VMEM((2, BLK_ROWS, ECOLS), jnp.int32),
            pltpu.VMEM((2, BLK_ROWS * ECOLS, FCW), jnp.float32),
            pltpu.VMEM_SHARED((NPAD, FCW), jnp.float32),
            pltpu.SemaphoreType.DMA((2,)),
            pltpu.SemaphoreType.DMA((2,)),
            pltpu.SemaphoreType.DMA,
            pltpu.SemaphoreType.DMA((2,)),
        ],
    )(hs_flat, src8, dst2d, zeros16)


def _tc_idx_body(src_ref, out_ref):
    fc = pl.program_id(0)
    out_ref[0] = src_ref[...] * 8 + fc


def _tc_idx(src2d):
    erb = 800
    return pl.pallas_call(
        _tc_idx_body, grid=(NFC, EROWS // erb),
        in_specs=[pl.BlockSpec((erb, ECOLS), lambda fc, j: (j, 0))],
        out_specs=pl.BlockSpec((1, erb, ECOLS), lambda fc, j: (fc, j, 0)),
        out_shape=jax.ShapeDtypeStruct((NFC, EROWS, ECOLS), jnp.int32),
    )(src2d)


def _tc_a_body(x_ref, cnt_ref, w1_ref, dinv_ref, hs_ref):
    cnt = cnt_ref[...]
    deg = cnt[:, 0:1] + cnt[:, 1:2] + 1.0
    dinv = lax.rsqrt(deg)
    dinv_ref[...] = dinv
    h = jnp.dot(x_ref[...], w1_ref[...], preferred_element_type=jnp.float32)
    hs_ref[...] = h * dinv


def _tc_a(x_pad, cnt_t, W1):
    nblk = NPAD // R
    grid = (nblk,)
    in_specs = [
        pl.BlockSpec((R, 64), lambda i: (i, 0)),
        pl.BlockSpec((R, 2), lambda i: (i, 0)),
        pl.BlockSpec((64, 128), lambda i: (0, 0)),
    ]
    out_specs = [pl.BlockSpec((R, 1), lambda i: (i, 0)),
                 pl.BlockSpec((R, 128), lambda i: (i, 0))]
    out_shape = [jax.ShapeDtypeStruct((NPAD, 1), jnp.float32),
                 jax.ShapeDtypeStruct((NPAD, 128), jnp.float32)]
    res = pl.pallas_call(
        _tc_a_body, grid=grid, in_specs=in_specs,
        out_specs=out_specs, out_shape=out_shape,
    )(x_pad, cnt_t, W1)
    return res[0], res[1]


def _tc_stats_body(s_ref, hs_ref, dinv_ref, params_ref, t_ref, sums_ref):
    i = pl.program_id(0)
    dinv = dinv_ref[...]
    b_row = params_ref[...][0:1, :]
    t = dinv * (s_ref[...] + hs_ref[...]) + b_row
    t_ref[...] = t
    node = i * R + lax.broadcasted_iota(jnp.int32, (R, 1), 0)
    mask = (node < N).astype(jnp.float32)
    tm = t * mask
    p0 = jnp.sum(tm, axis=0, keepdims=True)
    p1 = jnp.sum(tm * t, axis=0, keepdims=True)

    @pl.when(i == 0)
    def _():
        sums_ref[...] = jnp.zeros((8, 128), jnp.float32)

    sums_ref[0:1, :] += p0
    sums_ref[1:2, :] += p1


def _tc_stats(s, hs, dinv, b):
    nblk = NPAD // R
    params = jnp.zeros((8, 128), jnp.float32).at[0].set(b)
    in_specs = [pl.BlockSpec((R, 128), lambda i: (i, 0)),
                pl.BlockSpec((R, 128), lambda i: (i, 0)),
                pl.BlockSpec((R, 1), lambda i: (i, 0)),
                pl.BlockSpec((8, 128), lambda i: (0, 0))]
    out_specs = [pl.BlockSpec((R, 128), lambda i: (i, 0)),
                 pl.BlockSpec((8, 128), lambda i: (0, 0))]
    out_shape = [jax.ShapeDtypeStruct((NPAD, 128), jnp.float32),
                 jax.ShapeDtypeStruct((8, 128), jnp.float32)]
    return pl.pallas_call(
        _tc_stats_body, grid=(nblk,), in_specs=in_specs,
        out_specs=out_specs, out_shape=out_shape,
    )(s, hs, dinv, params)


def _bn_coefs(sums, params):
    mean = sums[0:1, :] / float(N)
    var = sums[1:2, :] / float(N) - mean * mean
    scale = params[0:1, :] * lax.rsqrt(var + 1e-5)
    shift = params[1:2, :] - mean * scale
    return scale, shift


def _tc_c_body(t_ref, sums_ref, dinv_ref, params_ref, w2_ref, hs_ref):
    scale, shift = _bn_coefs(sums_ref[...], params_ref[...])
    r = jnp.maximum(t_ref[...] * scale + shift, 0.0)
    h2 = jnp.dot(r, w2_ref[...], preferred_element_type=jnp.float32)
    hs_ref[...] = h2 * dinv_ref[...]


def _tc_c(t, sums, dinv, g, be, W2):
    nblk = NPAD // R
    params = jnp.zeros((8, 128), jnp.float32).at[0].set(g).at[1].set(be)
    in_specs = [
        pl.BlockSpec((R, 128), lambda i: (i, 0)),
        pl.BlockSpec((8, 128), lambda i: (0, 0)),
        pl.BlockSpec((R, 1), lambda i: (i, 0)),
        pl.BlockSpec((8, 128), lambda i: (0, 0)),
        pl.BlockSpec((128, 128), lambda i: (0, 0)),
    ]
    return pl.pallas_call(
        _tc_c_body, grid=(nblk,), in_specs=in_specs,
        out_specs=pl.BlockSpec((R, 128), lambda i: (i, 0)),
        out_shape=jax.ShapeDtypeStruct((NPAD, 128), jnp.float32),
    )(t, sums, dinv, params, W2)


def _tc_e_body(t_ref, sums_ref, params_ref, wh1_ref, y_ref):
    params = params_ref[...]
    scale, shift = _bn_coefs(sums_ref[...], params)
    r = jnp.maximum(t_ref[...] * scale + shift, 0.0)
    hh = jnp.dot(r, wh1_ref[...], preferred_element_type=jnp.float32)
    hh = jnp.maximum(hh + params[2:3, 0:64], 0.0)
    z = jnp.sum(hh * params[4:5, 0:64], axis=1, keepdims=True) \
        + params[3:4, 0:1]
    y_ref[...] = jax.nn.sigmoid(z)


def _tc_e(t2, sums2, g2, be2, Wh1, bh1, Wh2, bh2):
    nblk = NPAD // R
    params = (jnp.zeros((8, 128), jnp.float32)
              .at[0].set(g2).at[1].set(be2)
              .at[2, 0:64].set(bh1)
              .at[3, 0].set(bh2[0])
              .at[4, 0:64].set(Wh2[:, 0]))
    in_specs = [
        pl.BlockSpec((R, 128), lambda i: (i, 0)),
        pl.BlockSpec((8, 128), lambda i: (0, 0)),
        pl.BlockSpec((8, 128), lambda i: (0, 0)),
        pl.BlockSpec((128, 64), lambda i: (0, 0)),
    ]
    return pl.pallas_call(
        _tc_e_body, grid=(nblk,), in_specs=in_specs,
        out_specs=pl.BlockSpec((R, 1), lambda i: (i, 0)),
        out_shape=jax.ShapeDtypeStruct((NPAD, 1), jnp.float32),
    )(t2, sums2, params, Wh1)


def kernel(x, edge_index, W1, b1, g1, be1, W2, b2, g2, be2, Wh1, bh1, Wh2, bh2):
    ei = edge_index.astype(jnp.int32)
    src2d = ei[0].reshape(EROWS, ECOLS)
    dst2d = ei[1].reshape(EROWS, ECOLS)
    x_pad = jnp.pad(x, ((0, NPAD - N), (0, 0)))
    zeros1 = jnp.zeros((NPT,), jnp.float32)
    zeros16 = jnp.zeros((NPT, FCW), jnp.float32)

    src8 = _tc_idx(src2d)
    cnt = _sc_count(dst2d, zeros1)
    cnt_t = cnt.T
    dinv, hs1 = _tc_a(x_pad, cnt_t, W1)
    s1 = _sc_scatter(hs1.reshape(NPAD * NFC, FCW), src8, dst2d,
                     zeros16).reshape(NPAD, 128)
    t1, sums1 = _tc_stats(s1, hs1, dinv, b1)
    hs2 = _tc_c(t1, sums1, dinv, g1, be1, W2)
    s2 = _sc_scatter(hs2.reshape(NPAD * NFC, FCW), src8, dst2d,
                     zeros16).reshape(NPAD, 128)
    t2, sums2 = _tc_stats(s2, hs2, dinv, b2)
    y = _tc_e(t2, sums2, g2, be2, Wh1, bh1, Wh2, bh2)
    return y[:N]
